# baseline (speedup 1.0000x reference)
import jax
import jax.numpy as jnp
from jax import lax
from jax.experimental import pallas as pl
from jax.experimental.pallas import tpu as pltpu

D_MODEL = 4096
D_HIDDEN = 2048
NUM_EXPERTS = 64
Z_LOSS_COEF = 0.001
BALANCE_LOSS_COEF = 0.01

BLOCK_T = 512
CHUNK_H = 512
N_CHUNKS = D_HIDDEN // CHUNK_H


def _router_kernel(x_ref, w1_ref, b1_ref, w2_ref,
                   depth_ref, aux_ref,
                   logits_sc, probs_acc, counts_acc, lse2_acc):
    t = pl.program_id(0)
    nt = pl.num_programs(0)
    n_tokens = nt * BLOCK_T

    @pl.when(t == 0)
    def _init():
        probs_acc[...] = jnp.zeros_like(probs_acc)
        counts_acc[...] = jnp.zeros_like(counts_acc)
        lse2_acc[0, 0] = 0.0
        logits_sc[...] = jnp.zeros_like(logits_sc)

    prev_logits = logits_sc[...]

    def epilogue(logits, row, scale):
        m = jnp.max(logits, axis=1, keepdims=True)
        e = jnp.exp(logits - m)
        s = jnp.sum(e, axis=1, keepdims=True)
        probs = e / s
        lse = m + jnp.log(s)

        pm = jnp.max(probs, axis=1, keepdims=True)
        ii = lax.broadcasted_iota(jnp.int32, probs.shape, 1)
        sel = jnp.min(jnp.where(probs == pm, ii, NUM_EXPERTS), axis=1,
                      keepdims=True)
        depth_ref[pl.ds(row, 1), :, :] = jnp.reshape(sel[:, 0] + 1,
                                                     (1, 1, BLOCK_T))
        probs_acc[...] += scale * jnp.sum(probs, axis=0, keepdims=True)
        onehot = (ii == sel).astype(jnp.float32)
        counts_acc[...] += scale * jnp.sum(onehot, axis=0, keepdims=True)
        lse2_acc[0, 0] += scale * jnp.sum(lse * lse)

    epilogue(prev_logits, jnp.maximum(t - 1, 0),
             jnp.where(t > 0, 1.0, 0.0).astype(jnp.float32))

    xb = x_ref[...]
    logits = jnp.zeros((BLOCK_T, NUM_EXPERTS), jnp.float32)
    for c in range(N_CHUNKS):
        hc = jnp.dot(xb, w1_ref[c], preferred_element_type=jnp.float32)
        hc = hc + b1_ref[pl.ds(c, 1), :]
        hc = hc * jax.nn.sigmoid(hc)
        logits = logits + jnp.dot(
            hc, w2_ref[pl.ds(c * CHUNK_H, CHUNK_H), :],
            preferred_element_type=jnp.float32)
    logits_sc[...] = logits

    @pl.when(t == nt - 1)
    def _finalize():
        epilogue(logits_sc[...], t, jnp.float32(1.0))
        z_loss = lse2_acc[0, 0] / n_tokens
        bal = NUM_EXPERTS * jnp.sum(
            (counts_acc[...] / n_tokens) * (probs_acc[...] / n_tokens))
        aux_ref[0, 0] = Z_LOSS_COEF * z_loss + BALANCE_LOSS_COEF * bal


def kernel(x, W1, b1, W2):
    batch_size, seq_len, d_model = x.shape
    n_tokens = batch_size * seq_len
    nt = n_tokens // BLOCK_T
    x_flat = x.reshape(n_tokens, d_model)
    W1_c = W1.reshape(D_MODEL, N_CHUNKS, CHUNK_H).transpose(1, 0, 2)
    b1_c = b1.reshape(N_CHUNKS, CHUNK_H)

    depths, aux = pl.pallas_call(
        _router_kernel,
        grid=(nt,),
        in_specs=[
            pl.BlockSpec((BLOCK_T, D_MODEL), lambda t: (t, 0)),
            pl.BlockSpec((N_CHUNKS, D_MODEL, CHUNK_H), lambda t: (0, 0, 0)),
            pl.BlockSpec((N_CHUNKS, CHUNK_H), lambda t: (0, 0)),
            pl.BlockSpec((D_HIDDEN, NUM_EXPERTS), lambda t: (0, 0)),
        ],
        out_specs=[
            pl.BlockSpec((nt, 1, BLOCK_T), lambda t: (0, 0, 0)),
            pl.BlockSpec(memory_space=pltpu.SMEM),
        ],
        out_shape=[
            jax.ShapeDtypeStruct((nt, 1, BLOCK_T), jnp.int32),
            jax.ShapeDtypeStruct((1, 1), jnp.float32),
        ],
        scratch_shapes=[
            pltpu.VMEM((BLOCK_T, NUM_EXPERTS), jnp.float32),
            pltpu.VMEM((1, NUM_EXPERTS), jnp.float32),
            pltpu.VMEM((1, NUM_EXPERTS), jnp.float32),
            pltpu.SMEM((1, 1), jnp.float32),
        ],
        compiler_params=pltpu.CompilerParams(
            dimension_semantics=("arbitrary",),
        ),
    )(x_flat, W1_c, b1_c, W2)

    assigned_depths = depths.reshape(batch_size, seq_len)
    aux_loss = aux.reshape(())
    return assigned_depths, aux_loss


# speedup vs baseline: 1.1871x; 1.1871x over previous
import jax
import jax.numpy as jnp
from jax import lax
from jax.experimental import pallas as pl
from jax.experimental.pallas import tpu as pltpu

D_MODEL = 4096
D_HIDDEN = 2048
NUM_EXPERTS = 64
Z_LOSS_COEF = 0.001
BALANCE_LOSS_COEF = 0.01

BLOCK_T = 512


def _router_kernel(x_ref, w1_ref, b1_ref, w2_ref,
                   depth_ref, aux_ref,
                   logits_sc, probs_acc, counts_acc, lse2_acc):
    t = pl.program_id(0)
    nt = pl.num_programs(0)
    n_tokens = nt * BLOCK_T

    @pl.when(t == 0)
    def _init():
        probs_acc[...] = jnp.zeros_like(probs_acc)
        counts_acc[...] = jnp.zeros_like(counts_acc)
        lse2_acc[0, 0] = 0.0
        logits_sc[...] = jnp.zeros_like(logits_sc)

    prev_logits = logits_sc[...]

    def epilogue(logits, row, scale):
        m = jnp.max(logits, axis=1, keepdims=True)
        e = jnp.exp(logits - m)
        s = jnp.sum(e, axis=1, keepdims=True)
        probs = e / s
        lse = m + jnp.log(s)

        pm = jnp.max(probs, axis=1, keepdims=True)
        ii = lax.broadcasted_iota(jnp.int32, probs.shape, 1)
        sel = jnp.min(jnp.where(probs == pm, ii, NUM_EXPERTS), axis=1,
                      keepdims=True)
        depth_ref[pl.ds(row, 1), :, :] = jnp.reshape(sel[:, 0] + 1,
                                                     (1, 1, BLOCK_T))
        probs_acc[...] += scale * jnp.sum(probs, axis=0, keepdims=True)
        onehot = (ii == sel).astype(jnp.float32)
        counts_acc[...] += scale * jnp.sum(onehot, axis=0, keepdims=True)
        lse2_acc[0, 0] += scale * jnp.sum(lse * lse)

    epilogue(prev_logits, jnp.maximum(t - 1, 0),
             jnp.where(t > 0, 1.0, 0.0).astype(jnp.float32))

    xb = x_ref[...]
    h = jnp.dot(xb, w1_ref[...], preferred_element_type=jnp.float32)
    h = h + b1_ref[...]
    h = h * (0.5 * jnp.tanh(0.5 * h) + 0.5)
    logits_sc[...] = jnp.dot(h, w2_ref[...],
                             preferred_element_type=jnp.float32)

    @pl.when(t == nt - 1)
    def _finalize():
        epilogue(logits_sc[...], t, jnp.float32(1.0))
        z_loss = lse2_acc[0, 0] / n_tokens
        bal = NUM_EXPERTS * jnp.sum(
            (counts_acc[...] / n_tokens) * (probs_acc[...] / n_tokens))
        aux_ref[0, 0] = Z_LOSS_COEF * z_loss + BALANCE_LOSS_COEF * bal


def kernel(x, W1, b1, W2):
    batch_size, seq_len, d_model = x.shape
    n_tokens = batch_size * seq_len
    nt = n_tokens // BLOCK_T
    x_flat = x.reshape(n_tokens, d_model)
    b1_2d = b1.reshape(1, D_HIDDEN)

    depths, aux = pl.pallas_call(
        _router_kernel,
        grid=(nt,),
        in_specs=[
            pl.BlockSpec((BLOCK_T, D_MODEL), lambda t: (t, 0)),
            pl.BlockSpec((D_MODEL, D_HIDDEN), lambda t: (0, 0)),
            pl.BlockSpec((1, D_HIDDEN), lambda t: (0, 0)),
            pl.BlockSpec((D_HIDDEN, NUM_EXPERTS), lambda t: (0, 0)),
        ],
        out_specs=[
            pl.BlockSpec((nt, 1, BLOCK_T), lambda t: (0, 0, 0)),
            pl.BlockSpec(memory_space=pltpu.SMEM),
        ],
        out_shape=[
            jax.ShapeDtypeStruct((nt, 1, BLOCK_T), jnp.int32),
            jax.ShapeDtypeStruct((1, 1), jnp.float32),
        ],
        scratch_shapes=[
            pltpu.VMEM((BLOCK_T, NUM_EXPERTS), jnp.float32),
            pltpu.VMEM((1, NUM_EXPERTS), jnp.float32),
            pltpu.VMEM((1, NUM_EXPERTS), jnp.float32),
            pltpu.SMEM((1, 1), jnp.float32),
        ],
        compiler_params=pltpu.CompilerParams(
            dimension_semantics=("arbitrary",),
        ),
    )(x_flat, W1, b1_2d, W2)

    assigned_depths = depths.reshape(batch_size, seq_len)
    aux_loss = aux.reshape(())
    return assigned_depths, aux_loss


# speedup vs baseline: 1.1921x; 1.0042x over previous
import jax
import jax.numpy as jnp
from jax import lax
from jax.experimental import pallas as pl
from jax.experimental.pallas import tpu as pltpu

D_MODEL = 4096
D_HIDDEN = 2048
NUM_EXPERTS = 64
Z_LOSS_COEF = 0.001
BALANCE_LOSS_COEF = 0.01

BLOCK_T = 512


def _router_kernel(x_ref, w1_ref, b1_ref, w2_ref,
                   depth_ref, aux_ref,
                   logits_sc, probs_acc, counts_acc, lse2_acc):
    t = pl.program_id(0)
    nt = pl.num_programs(0)
    n_tokens = nt * BLOCK_T

    @pl.when(t == 0)
    def _init():
        probs_acc[...] = jnp.zeros_like(probs_acc)
        counts_acc[...] = jnp.zeros_like(counts_acc)
        lse2_acc[0, 0] = 0.0
        logits_sc[...] = jnp.zeros_like(logits_sc)

    prev_logits = logits_sc[...]

    def epilogue(logits, row, scale):
        m = jnp.max(logits, axis=1, keepdims=True)
        e = jnp.exp(logits - m)
        s = jnp.sum(e, axis=1, keepdims=True)
        probs = e / s
        lse = m + jnp.log(s)

        pm = jnp.max(probs, axis=1, keepdims=True)
        ii = lax.broadcasted_iota(jnp.int32, probs.shape, 1)
        sel = jnp.min(jnp.where(probs == pm, ii, NUM_EXPERTS), axis=1,
                      keepdims=True)
        depth_ref[pl.ds(row, 1), :, :] = jnp.reshape(sel[:, 0] + 1,
                                                     (1, 1, BLOCK_T))
        probs_acc[...] += scale * jnp.sum(probs, axis=0, keepdims=True)
        onehot = (ii == sel).astype(jnp.float32)
        counts_acc[...] += scale * jnp.sum(onehot, axis=0, keepdims=True)
        lse2_acc[0, 0] += scale * jnp.sum(lse * lse)

    epilogue(prev_logits, jnp.maximum(t - 1, 0),
             jnp.where(t > 0, 1.0, 0.0).astype(jnp.float32))

    xb = x_ref[...]
    h = jnp.dot(xb, w1_ref[...], preferred_element_type=jnp.float32)
    h = h + b1_ref[...]
    logits = jnp.zeros((BLOCK_T, NUM_EXPERTS), jnp.float32)
    ch = D_HIDDEN // 4
    for c in range(4):
        hc = h[:, c * ch:(c + 1) * ch]
        hc = hc * (0.5 * jnp.tanh(0.5 * hc) + 0.5)
        logits = logits + jnp.dot(hc, w2_ref[pl.ds(c * ch, ch), :],
                                  preferred_element_type=jnp.float32)
    logits_sc[...] = logits

    @pl.when(t == nt - 1)
    def _finalize():
        epilogue(logits_sc[...], t, jnp.float32(1.0))
        z_loss = lse2_acc[0, 0] / n_tokens
        bal = NUM_EXPERTS * jnp.sum(
            (counts_acc[...] / n_tokens) * (probs_acc[...] / n_tokens))
        aux_ref[0, 0] = Z_LOSS_COEF * z_loss + BALANCE_LOSS_COEF * bal


def kernel(x, W1, b1, W2):
    batch_size, seq_len, d_model = x.shape
    n_tokens = batch_size * seq_len
    nt = n_tokens // BLOCK_T
    x_flat = x.reshape(n_tokens, d_model)
    b1_2d = b1.reshape(1, D_HIDDEN)

    depths, aux = pl.pallas_call(
        _router_kernel,
        grid=(nt,),
        in_specs=[
            pl.BlockSpec((BLOCK_T, D_MODEL), lambda t: (t, 0)),
            pl.BlockSpec((D_MODEL, D_HIDDEN), lambda t: (0, 0)),
            pl.BlockSpec((1, D_HIDDEN), lambda t: (0, 0)),
            pl.BlockSpec((D_HIDDEN, NUM_EXPERTS), lambda t: (0, 0)),
        ],
        out_specs=[
            pl.BlockSpec((nt, 1, BLOCK_T), lambda t: (0, 0, 0)),
            pl.BlockSpec(memory_space=pltpu.SMEM),
        ],
        out_shape=[
            jax.ShapeDtypeStruct((nt, 1, BLOCK_T), jnp.int32),
            jax.ShapeDtypeStruct((1, 1), jnp.float32),
        ],
        scratch_shapes=[
            pltpu.VMEM((BLOCK_T, NUM_EXPERTS), jnp.float32),
            pltpu.VMEM((1, NUM_EXPERTS), jnp.float32),
            pltpu.VMEM((1, NUM_EXPERTS), jnp.float32),
            pltpu.SMEM((1, 1), jnp.float32),
        ],
        compiler_params=pltpu.CompilerParams(
            dimension_semantics=("arbitrary",),
        ),
    )(x_flat, W1, b1_2d, W2)

    assigned_depths = depths.reshape(batch_size, seq_len)
    aux_loss = aux.reshape(())
    return assigned_depths, aux_loss


# speedup vs baseline: 1.1944x; 1.0019x over previous
import jax
import jax.numpy as jnp
from jax import lax
from jax.experimental import pallas as pl
from jax.experimental.pallas import tpu as pltpu

D_MODEL = 4096
D_HIDDEN = 2048
NUM_EXPERTS = 64
Z_LOSS_COEF = 0.001
BALANCE_LOSS_COEF = 0.01

BLOCK_T = 512


def _router_kernel(x_ref, w1_ref, b1_ref, w2_ref,
                   depth_ref, aux_ref,
                   logits_sc, probs_acc, counts_acc, lse2_acc):
    t = pl.program_id(0)
    nt = pl.num_programs(0)
    n_tokens = nt * BLOCK_T

    @pl.when(t == 0)
    def _init():
        probs_acc[...] = jnp.zeros_like(probs_acc)
        counts_acc[...] = jnp.zeros_like(counts_acc)
        lse2_acc[0, 0] = 0.0
        logits_sc[...] = jnp.zeros_like(logits_sc)

    prev_logits = logits_sc[...]

    def epilogue(logits, row, scale):
        m = jnp.max(logits, axis=1, keepdims=True)
        e = jnp.exp(logits - m)
        s = jnp.sum(e, axis=1, keepdims=True)
        probs = e / s
        lse = m + jnp.log(s)

        pm = jnp.max(probs, axis=1, keepdims=True)
        ii = lax.broadcasted_iota(jnp.int32, probs.shape, 1)
        sel = jnp.min(jnp.where(probs == pm, ii, NUM_EXPERTS), axis=1,
                      keepdims=True)
        depth_ref[pl.ds(row, 1), :, :] = jnp.reshape(sel[:, 0] + 1,
                                                     (1, 1, BLOCK_T))
        probs_acc[...] += scale * jnp.sum(probs, axis=0, keepdims=True)
        onehot = (ii == sel).astype(jnp.float32)
        counts_acc[...] += scale * jnp.sum(onehot, axis=0, keepdims=True)
        lse2_acc[0, 0] += scale * jnp.sum(lse * lse)

    epilogue(prev_logits, jnp.maximum(t - 1, 0),
             jnp.where(t > 0, 1.0, 0.0).astype(jnp.float32))

    xb = x_ref[...]
    h = jnp.dot(xb, w1_ref[...], preferred_element_type=jnp.float32)
    h = h + b1_ref[...]
    h = h * (0.5 * jnp.tanh(0.5 * h) + 0.5)
    logits_sc[...] = jnp.dot(h, w2_ref[...],
                             preferred_element_type=jnp.float32)

    @pl.when(t == nt - 1)
    def _finalize():
        epilogue(logits_sc[...], t, jnp.float32(1.0))
        z_loss = lse2_acc[0, 0] / n_tokens
        bal = NUM_EXPERTS * jnp.sum(
            (counts_acc[...] / n_tokens) * (probs_acc[...] / n_tokens))
        aux_ref[0, 0] = Z_LOSS_COEF * z_loss + BALANCE_LOSS_COEF * bal


def kernel(x, W1, b1, W2):
    batch_size, seq_len, d_model = x.shape
    n_tokens = batch_size * seq_len
    nt = n_tokens // BLOCK_T
    x_flat = x.reshape(n_tokens, d_model)
    b1_2d = b1.reshape(1, D_HIDDEN)

    depths, aux = pl.pallas_call(
        _router_kernel,
        grid=(nt,),
        in_specs=[
            pl.BlockSpec((BLOCK_T, D_MODEL), lambda t: (t, 0)),
            pl.BlockSpec((D_MODEL, D_HIDDEN), lambda t: (0, 0)),
            pl.BlockSpec((1, D_HIDDEN), lambda t: (0, 0)),
            pl.BlockSpec((D_HIDDEN, NUM_EXPERTS), lambda t: (0, 0)),
        ],
        out_specs=[
            pl.BlockSpec((nt, 1, BLOCK_T), lambda t: (0, 0, 0)),
            pl.BlockSpec(memory_space=pltpu.SMEM),
        ],
        out_shape=[
            jax.ShapeDtypeStruct((nt, 1, BLOCK_T), jnp.int32),
            jax.ShapeDtypeStruct((1, 1), jnp.float32),
        ],
        scratch_shapes=[
            pltpu.VMEM((BLOCK_T, NUM_EXPERTS), jnp.float32),
            pltpu.VMEM((1, NUM_EXPERTS), jnp.float32),
            pltpu.VMEM((1, NUM_EXPERTS), jnp.float32),
            pltpu.SMEM((1, 1), jnp.float32),
        ],
        compiler_params=pltpu.CompilerParams(
            dimension_semantics=("arbitrary",),
        ),
    )(x_flat, W1, b1_2d, W2)

    assigned_depths = depths.reshape(batch_size, seq_len)
    aux_loss = aux.reshape(())
    return assigned_depths, aux_loss


# speedup vs baseline: 1.2269x; 1.0272x over previous
import jax
import jax.numpy as jnp
from jax import lax
from jax.experimental import pallas as pl
from jax.experimental.pallas import tpu as pltpu

D_MODEL = 4096
D_HIDDEN = 2048
NUM_EXPERTS = 64
Z_LOSS_COEF = 0.001
BALANCE_LOSS_COEF = 0.01

BLOCK_T = 512


def _router_kernel(x_ref, w1_ref, b1_ref, w2_ref,
                   depth_ref, aux_ref,
                   h_sc, probs_acc, counts_acc, lse2_acc):
    t = pl.program_id(0)
    nt = pl.num_programs(0)
    n_tokens = nt * BLOCK_T

    @pl.when(t == 0)
    def _init():
        probs_acc[...] = jnp.zeros_like(probs_acc)
        counts_acc[...] = jnp.zeros_like(counts_acc)
        lse2_acc[0, 0] = 0.0
        h_sc[...] = jnp.zeros_like(h_sc)

    def tail(h, row, scale):
        hs = h * (0.5 * jnp.tanh(0.5 * h) + 0.5)
        logits = jnp.dot(hs, w2_ref[...], preferred_element_type=jnp.float32)

        m = jnp.max(logits, axis=1, keepdims=True)
        e = jnp.exp(logits - m)
        s = jnp.sum(e, axis=1, keepdims=True)
        probs = e / s
        lse = m + jnp.log(s)

        pm = jnp.max(probs, axis=1, keepdims=True)
        ii = lax.broadcasted_iota(jnp.int32, probs.shape, 1)
        sel = jnp.min(jnp.where(probs == pm, ii, NUM_EXPERTS), axis=1,
                      keepdims=True)
        depth_ref[pl.ds(row, 1), :, :] = jnp.reshape(sel[:, 0] + 1,
                                                     (1, 1, BLOCK_T))
        probs_acc[...] += scale * jnp.sum(probs, axis=0, keepdims=True)
        onehot = (ii == sel).astype(jnp.float32)
        counts_acc[...] += scale * jnp.sum(onehot, axis=0, keepdims=True)
        lse2_acc[0, 0] += scale * jnp.sum(lse * lse)

    prev_h = h_sc[...]

    tail(prev_h, jnp.maximum(t - 1, 0),
         jnp.where(t > 0, 1.0, 0.0).astype(jnp.float32))

    xb = x_ref[...]
    h_sc[...] = jnp.dot(xb, w1_ref[...],
                        preferred_element_type=jnp.float32) + b1_ref[...]

    @pl.when(t == nt - 1)
    def _finalize():
        tail(h_sc[...], t, jnp.float32(1.0))
        z_loss = lse2_acc[0, 0] / n_tokens
        bal = NUM_EXPERTS * jnp.sum(
            (counts_acc[...] / n_tokens) * (probs_acc[...] / n_tokens))
        aux_ref[0, 0] = Z_LOSS_COEF * z_loss + BALANCE_LOSS_COEF * bal


def kernel(x, W1, b1, W2):
    batch_size, seq_len, d_model = x.shape
    n_tokens = batch_size * seq_len
    nt = n_tokens // BLOCK_T
    x_flat = x.reshape(n_tokens, d_model)
    b1_2d = b1.reshape(1, D_HIDDEN)

    depths, aux = pl.pallas_call(
        _router_kernel,
        grid=(nt,),
        in_specs=[
            pl.BlockSpec((BLOCK_T, D_MODEL), lambda t: (t, 0)),
            pl.BlockSpec((D_MODEL, D_HIDDEN), lambda t: (0, 0)),
            pl.BlockSpec((1, D_HIDDEN), lambda t: (0, 0)),
            pl.BlockSpec((D_HIDDEN, NUM_EXPERTS), lambda t: (0, 0)),
        ],
        out_specs=[
            pl.BlockSpec((nt, 1, BLOCK_T), lambda t: (0, 0, 0)),
            pl.BlockSpec(memory_space=pltpu.SMEM),
        ],
        out_shape=[
            jax.ShapeDtypeStruct((nt, 1, BLOCK_T), jnp.int32),
            jax.ShapeDtypeStruct((1, 1), jnp.float32),
        ],
        scratch_shapes=[
            pltpu.VMEM((BLOCK_T, D_HIDDEN), jnp.float32),
            pltpu.VMEM((1, NUM_EXPERTS), jnp.float32),
            pltpu.VMEM((1, NUM_EXPERTS), jnp.float32),
            pltpu.SMEM((1, 1), jnp.float32),
        ],
        compiler_params=pltpu.CompilerParams(
            dimension_semantics=("arbitrary",),
        ),
    )(x_flat, W1, b1_2d, W2)

    assigned_depths = depths.reshape(batch_size, seq_len)
    aux_loss = aux.reshape(())
    return assigned_depths, aux_loss
